# baseline (speedup 1.0000x reference)
import jax
import jax.numpy as jnp
from jax import lax
from jax.experimental import pallas as pl
from jax.experimental.pallas import tpu as pltpu
from jax.experimental.pallas import tpu_sc as plsc

N = 10000
NPAD = 10240
E = 320000
NW = 32
EPT = E // NW
CH = 128
NF = EPT // CH
TAIL = EPT - NF * CH
RPS = NPAD // 16
SCP = pltpu.CompilerParams(use_tc_tiling_on_sc=False)


def _stage_idx(dst_chunk, idx_all, base, n16):
    for k in range(n16):
        dst_chunk[pl.ds(16 * k, 16)] = idx_all[pl.ds(base + 16 * k, 16)]


def _deg_body(ei_hbm, ones_hbm, z_hbm, out_hbm, didx, ones_v, dchunk,
              dchunkT, vbuf, dv, acc):
    cid = lax.axis_index("c")
    sid = lax.axis_index("s")
    wid = cid * 16 + sid
    pltpu.sync_copy(ei_hbm.at[1, pl.ds(wid * EPT, EPT)], didx)
    pltpu.sync_copy(ones_hbm, ones_v)
    pltpu.sync_copy(z_hbm, acc.at[pl.ds(sid * RPS, RPS)])
    plsc.subcore_barrier()

    def body(j, carry):
        _stage_idx(dchunk, didx, j * CH, 8)
        pltpu.sync_copy(ones_v, acc.at[dchunk], add=True)
        return carry

    lax.fori_loop(0, NF, body, 0)
    _stage_idx(dchunkT, didx, NF * CH, 1)
    pltpu.sync_copy(ones_v.at[pl.ds(0, TAIL)], acc.at[dchunkT], add=True)
    plsc.subcore_barrier()
    pltpu.sync_copy(acc.at[pl.ds(sid * RPS, RPS)], vbuf)
    zero16 = jnp.zeros((16,), jnp.int32)
    row16 = lax.iota(jnp.int32, 16)

    def ext(i, carry):
        vals = plsc.load_gather(vbuf, [row16 + 16 * i, zero16])
        dv[pl.ds(16 * i, 16)] = vals
        return carry

    lax.fori_loop(0, RPS // 16, ext, 0)
    pltpu.sync_copy(dv, out_hbm.at[pl.ds(cid * NPAD + sid * RPS, RPS)])


def _deg_partials(ei):
    mesh = plsc.VectorSubcoreMesh(core_axis_name="c", subcore_axis_name="s")
    fn = pl.kernel(
        _deg_body,
        out_type=jax.ShapeDtypeStruct((2 * NPAD,), jnp.float32),
        mesh=mesh,
        scratch_types=[
            pltpu.VMEM((EPT,), jnp.int32),
            pltpu.VMEM((CH, 8), jnp.float32),
            pltpu.VMEM((CH,), jnp.int32),
            pltpu.VMEM((TAIL,), jnp.int32),
            pltpu.VMEM((RPS, 8), jnp.float32),
            pltpu.VMEM((RPS,), jnp.float32),
            pltpu.VMEM_SHARED((NPAD, 8), jnp.float32),
        ],
        compiler_params=pltpu.CompilerParams(
            use_tc_tiling_on_sc=False, needs_layout_passes=False),
    )
    return fn(ei, jnp.ones((CH, 8), jnp.float32), jnp.zeros((RPS, 8), jnp.float32))


def _edge_body(feat_hbm, ei_hbm, z_hbm, out_hbm,
               sidx, didx, dchunk0, dchunk1, dchunkT,
               rows0, rows1, rows2, rows3, rowsT,
               gs0, gs1, gs2, gs3, ss0, ss1, ss2, ss3, acc):
    cid = lax.axis_index("c")
    sid = lax.axis_index("s")
    wid = cid * 16 + sid
    base = wid * EPT
    pltpu.sync_copy(ei_hbm.at[0, pl.ds(base, EPT)], sidx)
    pltpu.sync_copy(ei_hbm.at[1, pl.ds(base, EPT)], didx)
    pltpu.sync_copy(z_hbm, acc.at[pl.ds(sid * RPS, RPS)])
    plsc.subcore_barrier()

    rows = (rows0, rows1, rows2, rows3)
    gsem = (gs0, gs1, gs2, gs3)
    ssem = (ss0, ss1, ss2, ss3)
    dch = (dchunk0, dchunk1)

    def gissue(j, b):
        pltpu.async_copy(feat_hbm.at[sidx.at[pl.ds(j * CH, CH)]],
                         rows[b], gsem[b])

    def gwait(j, b):
        pltpu.make_async_copy(feat_hbm.at[sidx.at[pl.ds(j * CH, CH)]],
                              rows[b], gsem[b]).wait()

    def sissue(b, dc):
        pltpu.async_copy(rows[b], acc.at[dc], ssem[b], add=True)

    def swait(b, dc):
        pltpu.make_async_copy(rows[b], acc.at[dc], ssem[b]).wait()

    gissue(0, 0)
    gissue(1, 1)

    def body(u, carry):
        j0 = 4 * u
        for k in range(4):
            j = j0 + k
            dc = dch[k % 2]
            dcp = dch[(k + 1) % 2]
            _stage_idx(dc, didx, j * CH, 8)
            gwait(j, k)
            sissue(k, dc)
            if k == 0:
                @pl.when(u > 0)
                def _():
                    swait(3, dcp)
            else:
                swait(k - 1, dcp)
            gissue(j + 2, (k + 2) % 4)
        return carry

    lax.fori_loop(0, NF // 4, body, 0)
    _stage_idx(dchunk0, didx, (NF - 2) * CH, 8)
    gwait(NF - 2, 0)
    sissue(0, dchunk0)
    swait(3, dchunk1)
    _stage_idx(dchunk1, didx, (NF - 1) * CH, 8)
    gwait(NF - 1, 1)
    sissue(1, dchunk1)
    swait(0, dchunk0)
    _stage_idx(dchunkT, didx, NF * CH, 1)
    pltpu.async_copy(
        feat_hbm.at[sidx.at[pl.ds(NF * CH, TAIL)]], rowsT, gs2).wait()
    pltpu.sync_copy(rowsT, acc.at[dchunkT], add=True)
    swait(1, dchunk1)
    plsc.subcore_barrier()
    pltpu.sync_copy(acc.at[pl.ds(sid * RPS, RPS)],
                    out_hbm.at[pl.ds(cid * NPAD + sid * RPS, RPS)])


def _edge_partials(feat, ei, d):
    mesh = plsc.VectorSubcoreMesh(core_axis_name="c", subcore_axis_name="s")
    fn = pl.kernel(
        _edge_body,
        out_type=jax.ShapeDtypeStruct((2 * NPAD, d), jnp.float32),
        mesh=mesh,
        scratch_types=[
            pltpu.VMEM((EPT,), jnp.int32),
            pltpu.VMEM((EPT,), jnp.int32),
            pltpu.VMEM((CH,), jnp.int32),
            pltpu.VMEM((CH,), jnp.int32),
            pltpu.VMEM((TAIL,), jnp.int32),
            pltpu.VMEM((CH, d), jnp.float32),
            pltpu.VMEM((CH, d), jnp.float32),
            pltpu.VMEM((CH, d), jnp.float32),
            pltpu.VMEM((CH, d), jnp.float32),
            pltpu.VMEM((TAIL, d), jnp.float32),
            pltpu.SemaphoreType.DMA,
            pltpu.SemaphoreType.DMA,
            pltpu.SemaphoreType.DMA,
            pltpu.SemaphoreType.DMA,
            pltpu.SemaphoreType.DMA,
            pltpu.SemaphoreType.DMA,
            pltpu.SemaphoreType.DMA,
            pltpu.SemaphoreType.DMA,
            pltpu.VMEM_SHARED((NPAD, d), jnp.float32),
        ],
        compiler_params=SCP,
    )
    return fn(feat, ei, jnp.zeros((RPS, d), jnp.float32))


def _dinv_col(p_ref):
    dinv = lax.rsqrt(p_ref[0:1, 0:NPAD] + p_ref[0:1, NPAD:2 * NPAD] + 1.0)
    return dinv.reshape(NPAD, 1)


def _denseH_body(x_ref, w1_ref, h_ref):
    h_ref[...] = jnp.dot(x_ref[...], w1_ref[...],
                         preferred_element_type=jnp.float32)


def _denseH(x, w1):
    return pl.pallas_call(
        _denseH_body,
        grid=(1,),
        in_specs=[
            pl.BlockSpec((NPAD, 128), lambda i: (0, 0)),
            pl.BlockSpec((128, 16), lambda i: (0, 0)),
        ],
        out_specs=pl.BlockSpec((NPAD, 16), lambda i: (0, 0)),
        out_shape=jax.ShapeDtypeStruct((NPAD, 16), jnp.float32),
    )(x, w1)


def _dense1_body(p_ref, h_ref, hs_ref):
    hs_ref[...] = h_ref[...] * _dinv_col(p_ref)


def _dense1(degp, h):
    return pl.pallas_call(
        _dense1_body,
        grid=(1,),
        in_specs=[
            pl.BlockSpec((1, 2 * NPAD), lambda i: (0, 0)),
            pl.BlockSpec((NPAD, 16), lambda i: (0, 0)),
        ],
        out_specs=pl.BlockSpec((NPAD, 16), lambda i: (0, 0)),
        out_shape=jax.ShapeDtypeStruct((NPAD, 16), jnp.float32),
    )(degp, h)


def _dense2_body(p_ref, q0_ref, q1_ref, hs_ref, b1_ref, w2_ref, gs_ref):
    dinv = _dinv_col(p_ref)
    o1 = dinv * (q0_ref[...] + q1_ref[...] + hs_ref[...]) + b1_ref[...]
    o1 = jnp.maximum(o1, 0.0)
    g = jnp.dot(o1, w2_ref[...], preferred_element_type=jnp.float32)
    gs_ref[...] = g * dinv


def _dense2(degp, q, hs, b1, w2p):
    return pl.pallas_call(
        _dense2_body,
        grid=(1,),
        in_specs=[
            pl.BlockSpec((1, 2 * NPAD), lambda i: (0, 0)),
            pl.BlockSpec((NPAD, 16), lambda i: (0, 0)),
            pl.BlockSpec((NPAD, 16), lambda i: (1, 0)),
            pl.BlockSpec((NPAD, 16), lambda i: (0, 0)),
            pl.BlockSpec((1, 16), lambda i: (0, 0)),
            pl.BlockSpec((16, 8), lambda i: (0, 0)),
        ],
        out_specs=pl.BlockSpec((NPAD, 8), lambda i: (0, 0)),
        out_shape=jax.ShapeDtypeStruct((NPAD, 8), jnp.float32),
    )(degp, q, q, hs, b1, w2p)


def _dense3_body(p_ref, r0_ref, r1_ref, gs_ref, b2_ref, out_ref):
    dinv = _dinv_col(p_ref)
    o2 = dinv * (r0_ref[...] + r1_ref[...] + gs_ref[...]) + b2_ref[...]
    logits = o2[:, :7]
    m = jnp.max(logits, axis=1, keepdims=True)
    z = logits - m
    out_ref[...] = z - jnp.log(jnp.sum(jnp.exp(z), axis=1, keepdims=True))


def _dense3(degp, r, gs, b2p):
    return pl.pallas_call(
        _dense3_body,
        grid=(1,),
        in_specs=[
            pl.BlockSpec((1, 2 * NPAD), lambda i: (0, 0)),
            pl.BlockSpec((NPAD, 8), lambda i: (0, 0)),
            pl.BlockSpec((NPAD, 8), lambda i: (1, 0)),
            pl.BlockSpec((NPAD, 8), lambda i: (0, 0)),
            pl.BlockSpec((1, 8), lambda i: (0, 0)),
        ],
        out_specs=pl.BlockSpec((NPAD, 7), lambda i: (0, 0)),
        out_shape=jax.ShapeDtypeStruct((N, 7), jnp.float32),
    )(degp, r, r, gs, b2p)


@jax.jit
def kernel(x, edge_index, W1, b1, W2, b2):
    ei = edge_index.astype(jnp.int32)

    h = _denseH(x, W1)
    degp = _deg_partials(ei).reshape(1, 2 * NPAD)
    hs = _dense1(degp, h)

    q = _edge_partials(hs, ei, 16)
    w2p = jnp.zeros((16, 8), jnp.float32).at[:, :7].set(W2)
    gs = _dense2(degp, q, hs, b1.reshape(1, 16), w2p)

    r = _edge_partials(gs, ei, 8)
    b2p = jnp.zeros((1, 8), jnp.float32).at[0, :7].set(b2)
    return _dense3(degp, r, gs, b2p)


# speedup vs baseline: 1.0109x; 1.0109x over previous
import jax
import jax.numpy as jnp
from jax import lax
from jax.experimental import pallas as pl
from jax.experimental.pallas import tpu as pltpu
from jax.experimental.pallas import tpu_sc as plsc

N = 10000
NPAD = 10240
E = 320000
NW = 32
EPT = E // NW
CH = 128
NF = EPT // CH
TAIL = EPT - NF * CH
RPS = NPAD // 16
SCP = pltpu.CompilerParams(use_tc_tiling_on_sc=False)


def _stage_idx(dst_chunk, idx_all, base, n16):
    for k in range(n16):
        dst_chunk[pl.ds(16 * k, 16)] = idx_all[pl.ds(base + 16 * k, 16)]


def _deg_body(ei_hbm, ones_hbm, z_hbm, out_hbm, didx, ones_v, dchunk,
              dchunkT, vbuf, dv, acc):
    cid = lax.axis_index("c")
    sid = lax.axis_index("s")
    wid = cid * 16 + sid
    pltpu.sync_copy(ei_hbm.at[1, pl.ds(wid * EPT, EPT)], didx)
    pltpu.sync_copy(ones_hbm, ones_v)
    pltpu.sync_copy(z_hbm, acc.at[pl.ds(sid * RPS, RPS)])
    plsc.subcore_barrier()

    def body(j, carry):
        _stage_idx(dchunk, didx, j * CH, 8)
        pltpu.sync_copy(ones_v, acc.at[dchunk], add=True)
        return carry

    lax.fori_loop(0, NF, body, 0)
    _stage_idx(dchunkT, didx, NF * CH, 1)
    pltpu.sync_copy(ones_v.at[pl.ds(0, TAIL)], acc.at[dchunkT], add=True)
    plsc.subcore_barrier()
    pltpu.sync_copy(acc.at[pl.ds(sid * RPS, RPS)], vbuf)
    zero16 = jnp.zeros((16,), jnp.int32)
    row16 = lax.iota(jnp.int32, 16)

    def ext(i, carry):
        vals = plsc.load_gather(vbuf, [row16 + 16 * i, zero16])
        dv[pl.ds(16 * i, 16)] = vals
        return carry

    lax.fori_loop(0, RPS // 16, ext, 0)
    pltpu.sync_copy(dv, out_hbm.at[pl.ds(cid * NPAD + sid * RPS, RPS)])


def _deg_partials(ei):
    mesh = plsc.VectorSubcoreMesh(core_axis_name="c", subcore_axis_name="s")
    fn = pl.kernel(
        _deg_body,
        out_type=jax.ShapeDtypeStruct((2 * NPAD,), jnp.float32),
        mesh=mesh,
        scratch_types=[
            pltpu.VMEM((EPT,), jnp.int32),
            pltpu.VMEM((CH, 8), jnp.float32),
            pltpu.VMEM((CH,), jnp.int32),
            pltpu.VMEM((TAIL,), jnp.int32),
            pltpu.VMEM((RPS, 8), jnp.float32),
            pltpu.VMEM((RPS,), jnp.float32),
            pltpu.VMEM_SHARED((NPAD, 8), jnp.float32),
        ],
        compiler_params=pltpu.CompilerParams(
            use_tc_tiling_on_sc=False, needs_layout_passes=False),
    )
    return fn(ei, jnp.ones((CH, 8), jnp.float32), jnp.zeros((RPS, 8), jnp.float32))


def _edge_body(feat_hbm, ei_hbm, z_hbm, out_hbm,
               sidx, didx, dchunk0, dchunk1, dchunkT,
               rows0, rows1, rows2, rows3, rowsT,
               gs0, gs1, gs2, gs3, ss0, ss1, ss2, ss3, acc):
    cid = lax.axis_index("c")
    sid = lax.axis_index("s")
    wid = cid * 16 + sid
    base = wid * EPT
    pltpu.sync_copy(ei_hbm.at[0, pl.ds(base, EPT)], sidx)
    pltpu.sync_copy(ei_hbm.at[1, pl.ds(base, EPT)], didx)
    pltpu.sync_copy(z_hbm, acc.at[pl.ds(sid * RPS, RPS)])
    plsc.subcore_barrier()

    rows = (rows0, rows1, rows2, rows3)
    gsem = (gs0, gs1, gs2, gs3)
    ssem = (ss0, ss1, ss2, ss3)
    dch = (dchunk0, dchunk1)

    def gissue(j, b):
        pltpu.async_copy(feat_hbm.at[sidx.at[pl.ds(j * CH, CH)]],
                         rows[b], gsem[b])

    def gwait(j, b):
        pltpu.make_async_copy(feat_hbm.at[sidx.at[pl.ds(j * CH, CH)]],
                              rows[b], gsem[b]).wait()

    def sissue(b, dc):
        pltpu.async_copy(rows[b], acc.at[dc], ssem[b], add=True)

    def swait(b, dc):
        pltpu.make_async_copy(rows[b], acc.at[dc], ssem[b]).wait()

    gissue(0, 0)
    gissue(1, 1)

    def body(u, carry):
        j0 = 4 * u
        for k in range(4):
            j = j0 + k
            dc = dch[k % 2]
            dcp = dch[(k + 1) % 2]
            _stage_idx(dc, didx, j * CH, 8)
            gwait(j, k)
            sissue(k, dc)
            if k == 0:
                @pl.when(u > 0)
                def _():
                    swait(3, dcp)
            else:
                swait(k - 1, dcp)
            gissue(j + 2, (k + 2) % 4)
        return carry

    lax.fori_loop(0, NF // 4, body, 0)
    _stage_idx(dchunk0, didx, (NF - 2) * CH, 8)
    gwait(NF - 2, 0)
    sissue(0, dchunk0)
    swait(3, dchunk1)
    _stage_idx(dchunk1, didx, (NF - 1) * CH, 8)
    gwait(NF - 1, 1)
    sissue(1, dchunk1)
    swait(0, dchunk0)
    _stage_idx(dchunkT, didx, NF * CH, 1)
    pltpu.async_copy(
        feat_hbm.at[sidx.at[pl.ds(NF * CH, TAIL)]], rowsT, gs2).wait()
    pltpu.sync_copy(rowsT, acc.at[dchunkT], add=True)
    swait(1, dchunk1)
    plsc.subcore_barrier()
    pltpu.sync_copy(acc.at[pl.ds(sid * RPS, RPS)],
                    out_hbm.at[pl.ds(cid * NPAD + sid * RPS, RPS)])


def _edge_partials(feat, ei, d):
    mesh = plsc.VectorSubcoreMesh(core_axis_name="c", subcore_axis_name="s")
    fn = pl.kernel(
        _edge_body,
        out_type=jax.ShapeDtypeStruct((2 * NPAD, d), jnp.float32),
        mesh=mesh,
        scratch_types=[
            pltpu.VMEM((EPT,), jnp.int32),
            pltpu.VMEM((EPT,), jnp.int32),
            pltpu.VMEM((CH,), jnp.int32),
            pltpu.VMEM((CH,), jnp.int32),
            pltpu.VMEM((TAIL,), jnp.int32),
            pltpu.VMEM((CH, d), jnp.float32),
            pltpu.VMEM((CH, d), jnp.float32),
            pltpu.VMEM((CH, d), jnp.float32),
            pltpu.VMEM((CH, d), jnp.float32),
            pltpu.VMEM((TAIL, d), jnp.float32),
            pltpu.SemaphoreType.DMA,
            pltpu.SemaphoreType.DMA,
            pltpu.SemaphoreType.DMA,
            pltpu.SemaphoreType.DMA,
            pltpu.SemaphoreType.DMA,
            pltpu.SemaphoreType.DMA,
            pltpu.SemaphoreType.DMA,
            pltpu.SemaphoreType.DMA,
            pltpu.VMEM_SHARED((NPAD, d), jnp.float32),
        ],
        compiler_params=SCP,
    )
    return fn(feat, ei, jnp.zeros((RPS, d), jnp.float32))


def _dinv_col(p_ref):
    dinv = lax.rsqrt(p_ref[0:1, 0:NPAD] + p_ref[0:1, NPAD:2 * NPAD] + 1.0)
    return dinv.reshape(NPAD, 1)


def _dense1_body(p_ref, x_ref, w1_ref, hs_ref):
    h = jnp.dot(x_ref[...], w1_ref[...], preferred_element_type=jnp.float32)
    hs_ref[...] = h * _dinv_col(p_ref)


def _dense1(degp, x, w1):
    return pl.pallas_call(
        _dense1_body,
        grid=(1,),
        in_specs=[
            pl.BlockSpec((1, 2 * NPAD), lambda i: (0, 0)),
            pl.BlockSpec((NPAD, 128), lambda i: (0, 0)),
            pl.BlockSpec((128, 16), lambda i: (0, 0)),
        ],
        out_specs=pl.BlockSpec((NPAD, 16), lambda i: (0, 0)),
        out_shape=jax.ShapeDtypeStruct((NPAD, 16), jnp.float32),
    )(degp, x, w1)


def _dense2_body(p_ref, q0_ref, q1_ref, hs_ref, b1_ref, w2_ref, gs_ref):
    dinv = _dinv_col(p_ref)
    o1 = dinv * (q0_ref[...] + q1_ref[...] + hs_ref[...]) + b1_ref[...]
    o1 = jnp.maximum(o1, 0.0)
    g = jnp.dot(o1, w2_ref[...], preferred_element_type=jnp.float32)
    gs_ref[...] = g * dinv


def _dense2(degp, q, hs, b1, w2p):
    return pl.pallas_call(
        _dense2_body,
        grid=(1,),
        in_specs=[
            pl.BlockSpec((1, 2 * NPAD), lambda i: (0, 0)),
            pl.BlockSpec((NPAD, 16), lambda i: (0, 0)),
            pl.BlockSpec((NPAD, 16), lambda i: (1, 0)),
            pl.BlockSpec((NPAD, 16), lambda i: (0, 0)),
            pl.BlockSpec((1, 16), lambda i: (0, 0)),
            pl.BlockSpec((16, 8), lambda i: (0, 0)),
        ],
        out_specs=pl.BlockSpec((NPAD, 8), lambda i: (0, 0)),
        out_shape=jax.ShapeDtypeStruct((NPAD, 8), jnp.float32),
    )(degp, q, q, hs, b1, w2p)


def _dense3_body(p_ref, r0_ref, r1_ref, gs_ref, b2_ref, out_ref):
    dinv = _dinv_col(p_ref)
    o2 = dinv * (r0_ref[...] + r1_ref[...] + gs_ref[...]) + b2_ref[...]
    logits = o2[:, :7]
    m = jnp.max(logits, axis=1, keepdims=True)
    z = logits - m
    out_ref[...] = z - jnp.log(jnp.sum(jnp.exp(z), axis=1, keepdims=True))


def _dense3(degp, r, gs, b2p):
    return pl.pallas_call(
        _dense3_body,
        grid=(1,),
        in_specs=[
            pl.BlockSpec((1, 2 * NPAD), lambda i: (0, 0)),
            pl.BlockSpec((NPAD, 8), lambda i: (0, 0)),
            pl.BlockSpec((NPAD, 8), lambda i: (1, 0)),
            pl.BlockSpec((NPAD, 8), lambda i: (0, 0)),
            pl.BlockSpec((1, 8), lambda i: (0, 0)),
        ],
        out_specs=pl.BlockSpec((NPAD, 7), lambda i: (0, 0)),
        out_shape=jax.ShapeDtypeStruct((N, 7), jnp.float32),
    )(degp, r, r, gs, b2p)


@jax.jit
def kernel(x, edge_index, W1, b1, W2, b2):
    ei = edge_index.astype(jnp.int32)

    degp = _deg_partials(ei).reshape(1, 2 * NPAD)
    hs = _dense1(degp, x, W1)

    q = _edge_partials(hs, ei, 16)
    w2p = jnp.zeros((16, 8), jnp.float32).at[:, :7].set(W2)
    gs = _dense2(degp, q, hs, b1.reshape(1, 16), w2p)

    r = _edge_partials(gs, ei, 8)
    b2p = jnp.zeros((1, 8), jnp.float32).at[0, :7].set(b2)
    return _dense3(degp, r, gs, b2p)
